# baseline (speedup 1.0000x reference)
import jax
import jax.numpy as jnp
from jax import lax
from jax.experimental import pallas as pl
from jax.experimental.pallas import tpu as pltpu
from jax.experimental.pallas import tpu_sc as plsc

_C = 160
_NBUF = 4
_NW = 32
_BT = 2000


def _sc_onehot(idx_hbm, attr_out, idx_all, bufs, sems):
    n = idx_hbm.shape[0]
    k = attr_out.shape[1]
    ck = _C * k
    num_chunks = n // _C
    nbase = num_chunks // _NW
    rem = num_chunks - nbase * _NW
    wid = lax.axis_index("s") * 2 + lax.axis_index("c")

    my_chunks = nbase + jnp.where(wid < rem, 1, 0)
    chunk0 = nbase * wid + jnp.minimum(wid, rem)
    atom0 = chunk0 * _C

    pltpu.sync_copy(idx_hbm.at[pl.ds(atom0, nbase * _C)], idx_all.at[pl.ds(0, nbase * _C)])

    @pl.when(wid < rem)
    def _():
        pltpu.sync_copy(
            idx_hbm.at[pl.ds(atom0 + nbase * _C, _C)],
            idx_all.at[pl.ds(nbase * _C, _C)],
        )

    zeros = jnp.zeros((16,), jnp.float32)
    ones = jnp.full((16,), 1.0, jnp.float32)

    def zinit(i, _):
        for b in range(_NBUF):
            for c in range(k // 16):
                bufs[b][i, pl.ds(c * 16, 16)] = zeros
            bufs[b][i, pl.ds(k - 16, 16)] = zeros
        return 0

    lax.fori_loop(0, _C, zinit, 0)

    def scatter(buf, j, val):
        for g in range(_C // 16):
            iv = idx_all[pl.ds(j * _C + g * 16, 16)]
            rows = lax.iota(jnp.int32, 16) + g * 16
            plsc.store_scatter(buf, [rows, iv], val)

    def dma(b, j):
        return pltpu.make_async_copy(
            bufs[b], attr_out.at[pl.ds((chunk0 + j) * _C, _C)], sems[b]
        )

    n_outer = (nbase + _NBUF) // _NBUF

    def outer(o, _):
        for b in range(_NBUF):
            j = o * _NBUF + b

            @pl.when(j < my_chunks)
            def _():
                @pl.when(o >= 1)
                def _():
                    dma(b, 0).wait()
                    scatter(bufs[b], j - _NBUF, zeros)

                scatter(bufs[b], j, ones)
                dma(b, j).start()

        return 0

    lax.fori_loop(0, n_outer, outer, 0)

    for b in range(_NBUF):
        @pl.when(b < my_chunks)
        def _():
            dma(b, 0).wait()


def _tc_fea(idx_ref, whi_ref, wlo_ref, out_ref):
    idx = idx_ref[0, 0, :]
    iota = lax.broadcasted_iota(jnp.int32, (_BT, whi_ref.shape[0]), 1)
    oh = (idx[:, None] == iota).astype(jnp.bfloat16)
    out_ref[...] = jnp.dot(
        oh, whi_ref[...], preferred_element_type=jnp.float32
    ) + jnp.dot(oh, wlo_ref[...], preferred_element_type=jnp.float32)


@jax.jit
def kernel(atom_number, W_embed):
    n = atom_number.shape[0]
    k, d = W_embed.shape
    assert n % _C == 0 and n % _BT == 0

    mesh = plsc.VectorSubcoreMesh(
        core_axis_name="c", subcore_axis_name="s", num_cores=2, num_subcores=16
    )
    nbase = (n // _C) // _NW
    attr = pl.kernel(
        _sc_onehot,
        out_type=jax.ShapeDtypeStruct((n, k), jnp.float32),
        mesh=mesh,
        compiler_params=pltpu.CompilerParams(needs_layout_passes=False),
        scratch_types=[
            pltpu.VMEM(((nbase + 1) * _C,), jnp.int32),
            [pltpu.VMEM((_C, k), jnp.float32) for _ in range(_NBUF)],
            [pltpu.SemaphoreType.DMA for _ in range(_NBUF)],
        ],
    )(atom_number)

    w_hi = W_embed.astype(jnp.bfloat16)
    w_lo = (W_embed - w_hi.astype(jnp.float32)).astype(jnp.bfloat16)
    nb = n // _BT
    fea = pl.pallas_call(
        _tc_fea,
        grid=(nb,),
        in_specs=[
            pl.BlockSpec((1, 1, _BT), lambda i: (i, 0, 0)),
            pl.BlockSpec((k, d), lambda i: (0, 0)),
            pl.BlockSpec((k, d), lambda i: (0, 0)),
        ],
        out_specs=pl.BlockSpec((_BT, d), lambda i: (i, 0)),
        out_shape=jax.ShapeDtypeStruct((n, d), jnp.float32),
    )(atom_number.reshape(nb, 1, _BT), w_hi, w_lo)

    return attr, fea


# speedup vs baseline: 1.1901x; 1.1901x over previous
import jax
import jax.numpy as jnp
from jax import lax
from jax.experimental import pallas as pl
from jax.experimental.pallas import tpu as pltpu
from jax.experimental.pallas import tpu_sc as plsc

_C = 160
_NBUF = 4
_NW = 32
_BT = 10000


def _sc_onehot(idx_hbm, attr_out, idx_all, bufs, sems):
    n = idx_hbm.shape[0]
    k = attr_out.shape[1]
    ck = _C * k
    num_chunks = n // _C
    nbase = num_chunks // _NW
    rem = num_chunks - nbase * _NW
    wid = lax.axis_index("s") * 2 + lax.axis_index("c")

    my_chunks = nbase + jnp.where(wid < rem, 1, 0)
    chunk0 = nbase * wid + jnp.minimum(wid, rem)
    atom0 = chunk0 * _C

    pltpu.sync_copy(idx_hbm.at[pl.ds(atom0, nbase * _C)], idx_all.at[pl.ds(0, nbase * _C)])

    @pl.when(wid < rem)
    def _():
        pltpu.sync_copy(
            idx_hbm.at[pl.ds(atom0 + nbase * _C, _C)],
            idx_all.at[pl.ds(nbase * _C, _C)],
        )

    zeros = jnp.zeros((16,), jnp.float32)
    ones = jnp.full((16,), 1.0, jnp.float32)

    def zinit(i, _):
        for b in range(_NBUF):
            for c in range(k // 16):
                bufs[b][i, pl.ds(c * 16, 16)] = zeros
            bufs[b][i, pl.ds(k - 16, 16)] = zeros
        return 0

    lax.fori_loop(0, _C, zinit, 0)

    def scatter(buf, j, val):
        for g in range(_C // 16):
            iv = idx_all[pl.ds(j * _C + g * 16, 16)]
            rows = lax.iota(jnp.int32, 16) + g * 16
            plsc.store_scatter(buf, [rows, iv], val)

    def dma(b, j):
        return pltpu.make_async_copy(
            bufs[b], attr_out.at[pl.ds((chunk0 + j) * _C, _C)], sems[b]
        )

    n_outer = (nbase + _NBUF) // _NBUF

    def outer(o, _):
        for b in range(_NBUF):
            j = o * _NBUF + b

            @pl.when(j < my_chunks)
            def _():
                @pl.when(o >= 1)
                def _():
                    dma(b, 0).wait()
                    scatter(bufs[b], j - _NBUF, zeros)

                scatter(bufs[b], j, ones)
                dma(b, j).start()

        return 0

    lax.fori_loop(0, n_outer, outer, 0)

    for b in range(_NBUF):
        @pl.when(b < my_chunks)
        def _():
            dma(b, 0).wait()


def _tc_fea(idx_ref, whi_ref, wlo_ref, out_ref):
    idx = idx_ref[0, 0, :]
    iota = lax.broadcasted_iota(jnp.int32, (_BT, whi_ref.shape[0]), 1)
    oh = (idx[:, None] == iota).astype(jnp.bfloat16)
    out_ref[...] = jnp.dot(
        oh, whi_ref[...], preferred_element_type=jnp.float32
    ) + jnp.dot(oh, wlo_ref[...], preferred_element_type=jnp.float32)


@jax.jit
def kernel(atom_number, W_embed):
    n = atom_number.shape[0]
    k, d = W_embed.shape
    assert n % _C == 0 and n % _BT == 0

    mesh = plsc.VectorSubcoreMesh(
        core_axis_name="c", subcore_axis_name="s", num_cores=2, num_subcores=16
    )
    nbase = (n // _C) // _NW
    attr = pl.kernel(
        _sc_onehot,
        out_type=jax.ShapeDtypeStruct((n, k), jnp.float32),
        mesh=mesh,
        compiler_params=pltpu.CompilerParams(needs_layout_passes=False),
        scratch_types=[
            pltpu.VMEM(((nbase + 1) * _C,), jnp.int32),
            [pltpu.VMEM((_C, k), jnp.float32) for _ in range(_NBUF)],
            [pltpu.SemaphoreType.DMA for _ in range(_NBUF)],
        ],
    )(atom_number)

    w_hi = W_embed.astype(jnp.bfloat16)
    w_lo = (W_embed - w_hi.astype(jnp.float32)).astype(jnp.bfloat16)
    nb = n // _BT
    fea = pl.pallas_call(
        _tc_fea,
        grid=(nb,),
        in_specs=[
            pl.BlockSpec((1, 1, _BT), lambda i: (i, 0, 0)),
            pl.BlockSpec((k, d), lambda i: (0, 0)),
            pl.BlockSpec((k, d), lambda i: (0, 0)),
        ],
        out_specs=pl.BlockSpec((_BT, d), lambda i: (i, 0)),
        out_shape=jax.ShapeDtypeStruct((n, d), jnp.float32),
    )(atom_number.reshape(nb, 1, _BT), w_hi, w_lo)

    return attr, fea
